# Initial kernel scaffold; baseline (speedup 1.0000x reference)
import jax
import jax.numpy as jnp
from jax.experimental import pallas as pl


def kernel(data, edge_index, Wih, Whh, bih, bhh, W1, b1, W2, b2):
    raise NotImplementedError("write your pallas kernel here")


# speedup vs baseline: 2.0835x; 2.0835x over previous
import functools

import jax
import jax.numpy as jnp
from jax.experimental import pallas as pl


def _lstm_proj_body(x_ref, wih_ref, whh_ref, b_ref, w1_ref, o_ref, *, T, H):
    h = jnp.zeros((x_ref.shape[0], H), jnp.float32)
    c = jnp.zeros((x_ref.shape[0], H), jnp.float32)
    for t in range(T):
        x_t = x_ref[:, 2 * t:2 * t + 2]
        gates = (jnp.dot(x_t, wih_ref[...], preferred_element_type=jnp.float32)
                 + jnp.dot(h, whh_ref[...], preferred_element_type=jnp.float32)
                 + b_ref[...])
        i = jax.nn.sigmoid(gates[:, 0:H])
        f = jax.nn.sigmoid(gates[:, H:2 * H])
        g = jnp.tanh(gates[:, 2 * H:3 * H])
        o = jax.nn.sigmoid(gates[:, 3 * H:4 * H])
        c = f * c + i * g
        h = o * jnp.tanh(c)
    o_ref[...] = jnp.dot(h, w1_ref[...], preferred_element_type=jnp.float32)


def _lstm_proj(data, Wih, Whh, bih, bhh, W1):
    T, N, _ = data.shape
    H = Whh.shape[1]
    G = 4 * H
    F = W1.shape[1]
    xt = jnp.transpose(data, (1, 0, 2)).reshape(N, T * 2)
    bias = (bih + bhh).reshape(1, G)
    B = 4000
    body = functools.partial(_lstm_proj_body, T=T, H=H)
    return pl.pallas_call(
        body,
        grid=(N // B,),
        in_specs=[
            pl.BlockSpec((B, T * 2), lambda i: (i, 0)),
            pl.BlockSpec((2, G), lambda i: (0, 0)),
            pl.BlockSpec((H, G), lambda i: (0, 0)),
            pl.BlockSpec((1, G), lambda i: (0, 0)),
            pl.BlockSpec((H, F), lambda i: (0, 0)),
        ],
        out_specs=pl.BlockSpec((B, F), lambda i: (i, 0)),
        out_shape=jax.ShapeDtypeStruct((N, F), jnp.float32),
    )(xt, Wih.T, Whh.T, bias, W1)


def kernel(data, edge_index, Wih, Whh, bih, bhh, W1, b1, W2, b2):
    N = data.shape[1]
    xw = _lstm_proj(data, Wih, Whh, bih, bhh, W1)

    src = edge_index[0]
    dst = edge_index[1]
    deg = jnp.ones((N,), jnp.float32).at[dst].add(1.0)
    dinv = jax.lax.rsqrt(deg)

    z = xw * dinv[:, None]
    agg = jnp.zeros((N, z.shape[1]), jnp.float32).at[dst].add(z[src])
    x1 = jax.nn.relu(dinv[:, None] * (agg + z) + b1)

    y = (x1 @ W2)[:, 0] * dinv
    agg2 = jnp.zeros((N,), jnp.float32).at[dst].add(y[src])
    out2 = dinv * (agg2 + y) + b2[0]
    return jax.nn.softmax(out2[:, None], axis=1)


# speedup vs baseline: 14.5227x; 6.9705x over previous
import functools

import jax
import jax.numpy as jnp
from jax import lax
from jax.experimental import pallas as pl
from jax.experimental.pallas import tpu as pltpu
from jax.experimental.pallas import tpu_sc as plsc

_NC, _NS, _LANES = 2, 16, 16
_NW = _NC * _NS
_CH = 128
_OB, _KCH = 50, 8
_RPW = _OB * _KCH
_EPAD = _NW * _RPW * _CH
_NPAD = 100352
_SROWS = _NPAD // _NS

_F = 16
_NB = 10000


def _mesh():
    return plsc.VectorSubcoreMesh(core_axis_name="c", subcore_axis_name="s",
                                  num_cores=_NC, num_subcores=_NS)


def _fill_zero_rows(ref, nrows):
    def body(i, _):
        ref[pl.ds(i, 1), :] = jnp.zeros((1, ref.shape[1]), jnp.float32)
        return _
    lax.fori_loop(0, nrows, body, None)


def _fill_const_1d(ref, n, val):
    def body(i, _):
        ref[pl.ds(i * _LANES, _LANES)] = jnp.full((_LANES,), val, jnp.float32)
        return _
    lax.fori_loop(0, n // _LANES, body, None)


def _sc_degree(dst2d):
    zlen = 1568

    @functools.partial(
        pl.kernel,
        out_type=jax.ShapeDtypeStruct((_NC * _NPAD,), jnp.float32),
        mesh=_mesh(),
        compiler_params=pltpu.CompilerParams(use_tc_tiling_on_sc=False),
        scratch_types=[
            pltpu.VMEM((_KCH, _CH), jnp.int32),
            pltpu.VMEM((_CH,), jnp.float32),
            pltpu.VMEM((zlen,), jnp.float32),
            pltpu.VMEM_SHARED((_NPAD,), jnp.float32),
            pltpu.SemaphoreType.DMA,
        ],
    )
    def deg_kernel(dst_hbm, out_hbm, idx_v, ones_v, zr_v, acc_sh, sem):
        c = lax.axis_index("c")
        s = lax.axis_index("s")
        wid = s * _NC + c
        _fill_const_1d(ones_v, _CH, 1.0)
        _fill_const_1d(zr_v, zlen, 0.0)
        for k in range(_SROWS // zlen):
            pltpu.sync_copy(zr_v, acc_sh.at[pl.ds(s * _SROWS + k * zlen, zlen)])
        plsc.subcore_barrier()
        row0 = wid * _RPW

        def outer(ob, _):
            pltpu.sync_copy(dst_hbm.at[pl.ds(row0 + ob * _KCH, _KCH)], idx_v)
            descs = [
                pltpu.async_copy(ones_v, acc_sh.at[idx_v.at[j]], sem, add=True)
                for j in range(_KCH)
            ]
            for d in descs:
                d.wait()
            return _

        lax.fori_loop(0, _OB, outer, None)
        plsc.subcore_barrier()
        pltpu.sync_copy(acc_sh.at[pl.ds(s * _SROWS, _SROWS)],
                        out_hbm.at[pl.ds(c * _NPAD + s * _SROWS, _SROWS)])

    return deg_kernel(dst2d)


def _sc_edge_pass_rows(src2d, dst2d, z):
    zrows = 392

    @functools.partial(
        pl.kernel,
        out_type=jax.ShapeDtypeStruct((_NC * _NPAD, _F), jnp.float32),
        mesh=_mesh(),
        compiler_params=pltpu.CompilerParams(use_tc_tiling_on_sc=False),
        scratch_types=[
            pltpu.VMEM((_KCH, _CH), jnp.int32),
            pltpu.VMEM((_KCH, _CH), jnp.int32),
            pltpu.VMEM((_CH, _F), jnp.float32),
            pltpu.VMEM((_CH, _F), jnp.float32),
            pltpu.VMEM((zrows, _F), jnp.float32),
            pltpu.SemaphoreType.DMA,
            pltpu.SemaphoreType.DMA,
            pltpu.VMEM_SHARED((_NPAD, _F), jnp.float32),
        ],
    )
    def pass_kernel(src_hbm, dst_hbm, z_hbm, out_hbm,
                    isrc_v, idst_v, rows0_v, rows1_v, zr_v, sem0, sem1, acc_sh):
        c = lax.axis_index("c")
        s = lax.axis_index("s")
        wid = s * _NC + c
        _fill_zero_rows(zr_v, zrows)
        for k in range(_SROWS // zrows):
            pltpu.sync_copy(
                zr_v, acc_sh.at[pl.ds(s * _SROWS + k * zrows, zrows)])
        plsc.subcore_barrier()
        row0 = wid * _RPW
        rows = (rows0_v, rows1_v)
        sems = (sem0, sem1)

        def outer(ob, _):
            r = row0 + ob * _KCH
            pltpu.sync_copy(src_hbm.at[pl.ds(r, _KCH)], isrc_v)
            pltpu.sync_copy(dst_hbm.at[pl.ds(r, _KCH)], idst_v)
            g = pltpu.async_copy(z_hbm.at[isrc_v.at[0]], rows[0], sems[0])
            for j in range(_KCH):
                if j + 1 < _KCH:
                    gn = pltpu.async_copy(
                        z_hbm.at[isrc_v.at[j + 1]], rows[(j + 1) % 2],
                        sems[(j + 1) % 2])
                g.wait()
                pltpu.sync_copy(rows[j % 2], acc_sh.at[idst_v.at[j]], add=True)
                if j + 1 < _KCH:
                    g = gn
            return _

        lax.fori_loop(0, _OB, outer, None)
        plsc.subcore_barrier()
        pltpu.sync_copy(acc_sh.at[pl.ds(s * _SROWS, _SROWS)],
                        out_hbm.at[pl.ds(c * _NPAD + s * _SROWS, _SROWS)])

    return pass_kernel(src2d, dst2d, z)


def _sc_edge_pass_scalar(src2d, dst2d, y):
    zlen = 1568

    @functools.partial(
        pl.kernel,
        out_type=jax.ShapeDtypeStruct((_NC * _NPAD,), jnp.float32),
        mesh=_mesh(),
        compiler_params=pltpu.CompilerParams(use_tc_tiling_on_sc=False),
        scratch_types=[
            pltpu.VMEM((_KCH, _CH), jnp.int32),
            pltpu.VMEM((_KCH, _CH), jnp.int32),
            pltpu.VMEM((_CH,), jnp.float32),
            pltpu.VMEM((_CH,), jnp.float32),
            pltpu.VMEM((zlen,), jnp.float32),
            pltpu.SemaphoreType.DMA,
            pltpu.SemaphoreType.DMA,
            pltpu.VMEM_SHARED((_NPAD,), jnp.float32),
        ],
    )
    def pass_kernel(src_hbm, dst_hbm, y_hbm, out_hbm,
                    isrc_v, idst_v, rows0_v, rows1_v, zr_v, sem0, sem1, acc_sh):
        c = lax.axis_index("c")
        s = lax.axis_index("s")
        wid = s * _NC + c
        _fill_const_1d(zr_v, zlen, 0.0)
        for k in range(_SROWS // zlen):
            pltpu.sync_copy(zr_v, acc_sh.at[pl.ds(s * _SROWS + k * zlen, zlen)])
        plsc.subcore_barrier()
        row0 = wid * _RPW
        rows = (rows0_v, rows1_v)
        sems = (sem0, sem1)

        def outer(ob, _):
            r = row0 + ob * _KCH
            pltpu.sync_copy(src_hbm.at[pl.ds(r, _KCH)], isrc_v)
            pltpu.sync_copy(dst_hbm.at[pl.ds(r, _KCH)], idst_v)
            g = pltpu.async_copy(y_hbm.at[isrc_v.at[0]], rows[0], sems[0])
            for j in range(_KCH):
                if j + 1 < _KCH:
                    gn = pltpu.async_copy(
                        y_hbm.at[isrc_v.at[j + 1]], rows[(j + 1) % 2],
                        sems[(j + 1) % 2])
                g.wait()
                pltpu.sync_copy(rows[j % 2], acc_sh.at[idst_v.at[j]], add=True)
                if j + 1 < _KCH:
                    g = gn
            return _

        lax.fori_loop(0, _OB, outer, None)
        plsc.subcore_barrier()
        pltpu.sync_copy(acc_sh.at[pl.ds(s * _SROWS, _SROWS)],
                        out_hbm.at[pl.ds(c * _NPAD + s * _SROWS, _SROWS)])

    return pass_kernel(src2d, dst2d, y)


def _lstm_proj_body(x_ref, wih_ref, whh_ref, b_ref, w1_ref, o_ref,
                    h_ref, c_ref, *, T, H):
    t = pl.program_id(1)

    @pl.when(t == 0)
    def _():
        h_ref[...] = jnp.zeros_like(h_ref)
        c_ref[...] = jnp.zeros_like(c_ref)

    gates = (jnp.dot(x_ref[...], wih_ref[...],
                     preferred_element_type=jnp.float32)
             + jnp.dot(h_ref[...], whh_ref[...],
                       preferred_element_type=jnp.float32)
             + b_ref[...])
    i = jax.nn.sigmoid(gates[:, 0:H])
    f = jax.nn.sigmoid(gates[:, H:2 * H])
    g = jnp.tanh(gates[:, 2 * H:3 * H])
    o = jax.nn.sigmoid(gates[:, 3 * H:4 * H])
    c = f * c_ref[...] + i * g
    h = o * jnp.tanh(c)
    c_ref[...] = c
    h_ref[...] = h

    @pl.when(t == T - 1)
    def _():
        o_ref[...] = jnp.dot(h, w1_ref[...],
                             preferred_element_type=jnp.float32)


def _lstm_proj(data, Wih, Whh, bih, bhh, W1):
    T, N, _ = data.shape
    H = Whh.shape[1]
    G = 4 * H
    F = W1.shape[1]
    xt = data.reshape(T * N, 2)
    bias = (bih + bhh).reshape(1, G)
    B = 4000
    nb = N // B
    body = functools.partial(_lstm_proj_body, T=T, H=H)
    return pl.pallas_call(
        body,
        grid=(nb, T),
        in_specs=[
            pl.BlockSpec((B, 2), lambda i, t: (t * nb + i, 0)),
            pl.BlockSpec((2, G), lambda i, t: (0, 0)),
            pl.BlockSpec((H, G), lambda i, t: (0, 0)),
            pl.BlockSpec((1, G), lambda i, t: (0, 0)),
            pl.BlockSpec((H, F), lambda i, t: (0, 0)),
        ],
        out_specs=pl.BlockSpec((B, F), lambda i, t: (i, 0)),
        out_shape=jax.ShapeDtypeStruct((N, F), jnp.float32),
        scratch_shapes=[
            pltpu.VMEM((B, H), jnp.float32),
            pltpu.VMEM((B, H), jnp.float32),
        ],
    )(xt, Wih.T, Whh.T, bias, W1)


def _scale_body(degt_ref, xw_ref, z_ref, dinv_ref):
    d = 1.0 + degt_ref[:, 0:1] + degt_ref[:, 1:2]
    dinv = lax.rsqrt(d)
    z_ref[...] = xw_ref[...] * dinv
    dinv_ref[...] = dinv * jnp.ones((1, _F), jnp.float32)


def _scale(degpT, xw, N):
    return pl.pallas_call(
        _scale_body,
        grid=(N // _NB,),
        in_specs=[
            pl.BlockSpec((_NB, 2), lambda i: (i, 0)),
            pl.BlockSpec((_NB, _F), lambda i: (i, 0)),
        ],
        out_specs=[
            pl.BlockSpec((_NB, _F), lambda i: (i, 0)),
            pl.BlockSpec((_NB, _F), lambda i: (i, 0)),
        ],
        out_shape=[
            jax.ShapeDtypeStruct((N, _F), jnp.float32),
            jax.ShapeDtypeStruct((N, _F), jnp.float32),
        ],
    )(degpT, xw)


def _conv1_head_body(acc0_ref, acc1_ref, z_ref, dinv_ref, b1_ref, w2_ref, y_ref):
    a = acc0_ref[...] + acc1_ref[...] + z_ref[...]
    x1 = jnp.maximum(dinv_ref[...] * a + b1_ref[...], 0.0)
    y = jnp.sum(x1 * w2_ref[...], axis=1, keepdims=True)
    y_ref[...] = y * dinv_ref[:, 0:1]


def _conv1_head(acc1p0, acc1p1, z, dinv16, b1, W2, N):
    return pl.pallas_call(
        _conv1_head_body,
        grid=(N // _NB,),
        in_specs=[
            pl.BlockSpec((_NB, _F), lambda i: (i, 0)),
            pl.BlockSpec((_NB, _F), lambda i: (i, 0)),
            pl.BlockSpec((_NB, _F), lambda i: (i, 0)),
            pl.BlockSpec((_NB, _F), lambda i: (i, 0)),
            pl.BlockSpec((1, _F), lambda i: (0, 0)),
            pl.BlockSpec((1, _F), lambda i: (0, 0)),
        ],
        out_specs=pl.BlockSpec((_NB, 1), lambda i: (i, 0)),
        out_shape=jax.ShapeDtypeStruct((N, 1), jnp.float32),
    )(acc1p0, acc1p1, z, dinv16, b1.reshape(1, _F), W2.reshape(1, _F))


def _final_body(acc2t_ref, y_ref, dinv_ref, b2_ref, o_ref):
    t = (dinv_ref[:, 0:1]
         * (acc2t_ref[:, 0:1] + acc2t_ref[:, 1:2] + y_ref[...])
         + b2_ref[0, 0])
    m = jnp.max(t, axis=1, keepdims=True)
    e = jnp.exp(t - m)
    o_ref[...] = e / jnp.sum(e, axis=1, keepdims=True)


def _final(acc2pT, yprime, dinv16, b2, N):
    return pl.pallas_call(
        _final_body,
        grid=(N // _NB,),
        in_specs=[
            pl.BlockSpec((_NB, 2), lambda i: (i, 0)),
            pl.BlockSpec((_NB, 1), lambda i: (i, 0)),
            pl.BlockSpec((_NB, _F), lambda i: (i, 0)),
            pl.BlockSpec((1, 1), lambda i: (0, 0)),
        ],
        out_specs=pl.BlockSpec((_NB, 1), lambda i: (i, 0)),
        out_shape=jax.ShapeDtypeStruct((N, 1), jnp.float32),
    )(acc2pT, yprime, dinv16, b2.reshape(1, 1))


def kernel(data, edge_index, Wih, Whh, bih, bhh, W1, b1, W2, b2):
    N = data.shape[1]
    E = edge_index.shape[1]
    pad = _EPAD - E
    src2d = jnp.concatenate(
        [edge_index[0], jnp.zeros((pad,), jnp.int32)]).reshape(-1, _CH)
    dst2d = jnp.concatenate(
        [edge_index[1], jnp.full((pad,), N, jnp.int32)]).reshape(-1, _CH)

    deg = _sc_degree(dst2d)
    xw = _lstm_proj(data, Wih, Whh, bih, bhh, W1)

    degpT = deg.reshape(_NC, _NPAD).T
    z, dinv16 = _scale(degpT, xw, N)
    a1p = _sc_edge_pass_rows(src2d, dst2d, z)
    yprime = _conv1_head(a1p[:_NPAD], a1p[_NPAD:], z, dinv16, b1, W2, N)
    a2p = _sc_edge_pass_scalar(src2d, dst2d, yprime.reshape(N))
    acc2pT = a2p.reshape(_NC, _NPAD).T
    return _final(acc2pT, yprime, dinv16, b2, N)


# speedup vs baseline: 14.5377x; 1.0010x over previous
import functools

import jax
import jax.numpy as jnp
from jax import lax
from jax.experimental import pallas as pl
from jax.experimental.pallas import tpu as pltpu
from jax.experimental.pallas import tpu_sc as plsc

_NC, _NS, _LANES = 2, 16, 16
_NW = _NC * _NS
_CH = 128
_OB, _KCH = 50, 8
_RPW = _OB * _KCH
_EPAD = _NW * _RPW * _CH
_NPAD = 100352
_SROWS = _NPAD // _NS

_F = 16
_NB = 10000


def _mesh():
    return plsc.VectorSubcoreMesh(core_axis_name="c", subcore_axis_name="s",
                                  num_cores=_NC, num_subcores=_NS)


def _fill_zero_rows(ref, nrows):
    def body(i, _):
        ref[pl.ds(i, 1), :] = jnp.zeros((1, ref.shape[1]), jnp.float32)
        return _
    lax.fori_loop(0, nrows, body, None)


def _fill_const_1d(ref, n, val):
    def body(i, _):
        ref[pl.ds(i * _LANES, _LANES)] = jnp.full((_LANES,), val, jnp.float32)
        return _
    lax.fori_loop(0, n // _LANES, body, None)


def _sc_degree(dst2d):
    zlen = 1568

    @functools.partial(
        pl.kernel,
        out_type=jax.ShapeDtypeStruct((_NC * _NPAD,), jnp.float32),
        mesh=_mesh(),
        compiler_params=pltpu.CompilerParams(use_tc_tiling_on_sc=False),
        scratch_types=[
            pltpu.VMEM((_KCH, _CH), jnp.int32),
            pltpu.VMEM((_CH,), jnp.float32),
            pltpu.VMEM((zlen,), jnp.float32),
            pltpu.VMEM_SHARED((_NPAD,), jnp.float32),
            pltpu.SemaphoreType.DMA,
        ],
    )
    def deg_kernel(dst_hbm, out_hbm, idx_v, ones_v, zr_v, acc_sh, sem):
        c = lax.axis_index("c")
        s = lax.axis_index("s")
        wid = s * _NC + c
        _fill_const_1d(ones_v, _CH, 1.0)
        _fill_const_1d(zr_v, zlen, 0.0)
        for k in range(_SROWS // zlen):
            pltpu.sync_copy(zr_v, acc_sh.at[pl.ds(s * _SROWS + k * zlen, zlen)])
        plsc.subcore_barrier()
        row0 = wid * _RPW

        def outer(ob, _):
            pltpu.sync_copy(dst_hbm.at[pl.ds(row0 + ob * _KCH, _KCH)], idx_v)
            descs = [
                pltpu.async_copy(ones_v, acc_sh.at[idx_v.at[j]], sem, add=True)
                for j in range(_KCH)
            ]
            for d in descs:
                d.wait()
            return _

        lax.fori_loop(0, _OB, outer, None)
        plsc.subcore_barrier()
        pltpu.sync_copy(acc_sh.at[pl.ds(s * _SROWS, _SROWS)],
                        out_hbm.at[pl.ds(c * _NPAD + s * _SROWS, _SROWS)])

    return deg_kernel(dst2d)


def _sc_edge_pass_rows(src2d, dst2d, z):
    zrows = 392

    @functools.partial(
        pl.kernel,
        out_type=jax.ShapeDtypeStruct((_NC * _NPAD, _F), jnp.float32),
        mesh=_mesh(),
        compiler_params=pltpu.CompilerParams(use_tc_tiling_on_sc=False),
        scratch_types=[
            pltpu.VMEM((_KCH, _CH), jnp.int32),
            pltpu.VMEM((_KCH, _CH), jnp.int32),
            pltpu.VMEM((_CH, _F), jnp.float32),
            pltpu.VMEM((_CH, _F), jnp.float32),
            pltpu.VMEM((zrows, _F), jnp.float32),
            pltpu.SemaphoreType.DMA,
            pltpu.SemaphoreType.DMA,
            pltpu.VMEM_SHARED((_NPAD, _F), jnp.float32),
        ],
    )
    def pass_kernel(src_hbm, dst_hbm, z_hbm, out_hbm,
                    isrc_v, idst_v, rows0_v, rows1_v, zr_v, sem0, sem1, acc_sh):
        c = lax.axis_index("c")
        s = lax.axis_index("s")
        wid = s * _NC + c
        _fill_zero_rows(zr_v, zrows)
        for k in range(_SROWS // zrows):
            pltpu.sync_copy(
                zr_v, acc_sh.at[pl.ds(s * _SROWS + k * zrows, zrows)])
        plsc.subcore_barrier()
        row0 = wid * _RPW
        rows = (rows0_v, rows1_v)
        sems = (sem0, sem1)

        def outer(ob, _):
            r = row0 + ob * _KCH
            pltpu.sync_copy(src_hbm.at[pl.ds(r, _KCH)], isrc_v)
            pltpu.sync_copy(dst_hbm.at[pl.ds(r, _KCH)], idst_v)
            g = pltpu.async_copy(z_hbm.at[isrc_v.at[0]], rows[0], sems[0])
            for j in range(_KCH):
                if j + 1 < _KCH:
                    gn = pltpu.async_copy(
                        z_hbm.at[isrc_v.at[j + 1]], rows[(j + 1) % 2],
                        sems[(j + 1) % 2])
                g.wait()
                pltpu.sync_copy(rows[j % 2], acc_sh.at[idst_v.at[j]], add=True)
                if j + 1 < _KCH:
                    g = gn
            return _

        lax.fori_loop(0, _OB, outer, None)
        plsc.subcore_barrier()
        pltpu.sync_copy(acc_sh.at[pl.ds(s * _SROWS, _SROWS)],
                        out_hbm.at[pl.ds(c * _NPAD + s * _SROWS, _SROWS)])

    return pass_kernel(src2d, dst2d, z)


def _sc_edge_pass_scalar(src2d, dst2d, y):
    zlen = 1568

    @functools.partial(
        pl.kernel,
        out_type=jax.ShapeDtypeStruct((_NC * _NPAD,), jnp.float32),
        mesh=_mesh(),
        compiler_params=pltpu.CompilerParams(use_tc_tiling_on_sc=False),
        scratch_types=[
            pltpu.VMEM((_KCH, _CH), jnp.int32),
            pltpu.VMEM((_KCH, _CH), jnp.int32),
            pltpu.VMEM((_CH,), jnp.float32),
            pltpu.VMEM((_CH,), jnp.float32),
            pltpu.VMEM((zlen,), jnp.float32),
            pltpu.SemaphoreType.DMA,
            pltpu.SemaphoreType.DMA,
            pltpu.VMEM_SHARED((_NPAD,), jnp.float32),
        ],
    )
    def pass_kernel(src_hbm, dst_hbm, y_hbm, out_hbm,
                    isrc_v, idst_v, rows0_v, rows1_v, zr_v, sem0, sem1, acc_sh):
        c = lax.axis_index("c")
        s = lax.axis_index("s")
        wid = s * _NC + c
        _fill_const_1d(zr_v, zlen, 0.0)
        for k in range(_SROWS // zlen):
            pltpu.sync_copy(zr_v, acc_sh.at[pl.ds(s * _SROWS + k * zlen, zlen)])
        plsc.subcore_barrier()
        row0 = wid * _RPW
        rows = (rows0_v, rows1_v)
        sems = (sem0, sem1)

        def outer(ob, _):
            r = row0 + ob * _KCH
            pltpu.sync_copy(src_hbm.at[pl.ds(r, _KCH)], isrc_v)
            pltpu.sync_copy(dst_hbm.at[pl.ds(r, _KCH)], idst_v)
            g = pltpu.async_copy(y_hbm.at[isrc_v.at[0]], rows[0], sems[0])
            for j in range(_KCH):
                if j + 1 < _KCH:
                    gn = pltpu.async_copy(
                        y_hbm.at[isrc_v.at[j + 1]], rows[(j + 1) % 2],
                        sems[(j + 1) % 2])
                g.wait()
                pltpu.sync_copy(rows[j % 2], acc_sh.at[idst_v.at[j]], add=True)
                if j + 1 < _KCH:
                    g = gn
            return _

        lax.fori_loop(0, _OB, outer, None)
        plsc.subcore_barrier()
        pltpu.sync_copy(acc_sh.at[pl.ds(s * _SROWS, _SROWS)],
                        out_hbm.at[pl.ds(c * _NPAD + s * _SROWS, _SROWS)])

    return pass_kernel(src2d, dst2d, y)


def _lstm_proj_body(x_ref, wih_ref, whh_ref, b_ref, w1_ref, o_ref,
                    h_ref, c_ref, *, T, H):
    t = pl.program_id(1)

    @pl.when(t == 0)
    def _():
        h_ref[...] = jnp.zeros_like(h_ref)
        c_ref[...] = jnp.zeros_like(c_ref)

    gates = (jnp.dot(x_ref[...], wih_ref[...],
                     preferred_element_type=jnp.float32)
             + jnp.dot(h_ref[...], whh_ref[...],
                       preferred_element_type=jnp.float32)
             + b_ref[...])
    i = jax.nn.sigmoid(gates[:, 0:H])
    f = jax.nn.sigmoid(gates[:, H:2 * H])
    g = jnp.tanh(gates[:, 2 * H:3 * H])
    o = jax.nn.sigmoid(gates[:, 3 * H:4 * H])
    c = f * c_ref[...] + i * g
    h = o * jnp.tanh(c)
    c_ref[...] = c
    h_ref[...] = h

    @pl.when(t == T - 1)
    def _():
        o_ref[...] = jnp.dot(h, w1_ref[...],
                             preferred_element_type=jnp.float32)


def _lstm_proj(data, Wih, Whh, bih, bhh, W1):
    T, N, _ = data.shape
    H = Whh.shape[1]
    G = 4 * H
    F = W1.shape[1]
    xt = data.reshape(T * N, 2)
    bias = (bih + bhh).reshape(1, G)
    B = 4000
    nb = N // B
    body = functools.partial(_lstm_proj_body, T=T, H=H)
    return pl.pallas_call(
        body,
        grid=(nb, T),
        in_specs=[
            pl.BlockSpec((B, 2), lambda i, t: (t * nb + i, 0)),
            pl.BlockSpec((2, G), lambda i, t: (0, 0)),
            pl.BlockSpec((H, G), lambda i, t: (0, 0)),
            pl.BlockSpec((1, G), lambda i, t: (0, 0)),
            pl.BlockSpec((H, F), lambda i, t: (0, 0)),
        ],
        out_specs=pl.BlockSpec((B, F), lambda i, t: (i, 0)),
        out_shape=jax.ShapeDtypeStruct((N, F), jnp.float32),
        scratch_shapes=[
            pltpu.VMEM((B, H), jnp.float32),
            pltpu.VMEM((B, H), jnp.float32),
        ],
    )(xt, Wih.T, Whh.T, bias, W1)


def _scale_body(degt_ref, xw_ref, z_ref, dinv_ref):
    d = 1.0 + degt_ref[:, 0:1] + degt_ref[:, 1:2]
    dinv = lax.rsqrt(d)
    z_ref[...] = xw_ref[...] * dinv
    dinv_ref[...] = dinv * jnp.ones((1, _F), jnp.float32)


def _scale(degpT, xw, N):
    return pl.pallas_call(
        _scale_body,
        grid=(N // _NB,),
        in_specs=[
            pl.BlockSpec((_NB, 2), lambda i: (i, 0)),
            pl.BlockSpec((_NB, _F), lambda i: (i, 0)),
        ],
        out_specs=[
            pl.BlockSpec((_NB, _F), lambda i: (i, 0)),
            pl.BlockSpec((_NB, _F), lambda i: (i, 0)),
        ],
        out_shape=[
            jax.ShapeDtypeStruct((N, _F), jnp.float32),
            jax.ShapeDtypeStruct((N, _F), jnp.float32),
        ],
    )(degpT, xw)


def _conv1_head_body(acc0_ref, acc1_ref, z_ref, dinv_ref, b1_ref, w2_ref, y_ref):
    a = acc0_ref[...] + acc1_ref[...] + z_ref[...]
    x1 = jnp.maximum(dinv_ref[...] * a + b1_ref[...], 0.0)
    y = jnp.sum(x1 * w2_ref[...], axis=1, keepdims=True)
    y_ref[...] = y * dinv_ref[:, 0:1]


def _conv1_head(acc1p0, acc1p1, z, dinv16, b1, W2, N):
    return pl.pallas_call(
        _conv1_head_body,
        grid=(N // _NB,),
        in_specs=[
            pl.BlockSpec((_NB, _F), lambda i: (i, 0)),
            pl.BlockSpec((_NB, _F), lambda i: (i, 0)),
            pl.BlockSpec((_NB, _F), lambda i: (i, 0)),
            pl.BlockSpec((_NB, _F), lambda i: (i, 0)),
            pl.BlockSpec((1, _F), lambda i: (0, 0)),
            pl.BlockSpec((1, _F), lambda i: (0, 0)),
        ],
        out_specs=pl.BlockSpec((_NB, 1), lambda i: (i, 0)),
        out_shape=jax.ShapeDtypeStruct((N, 1), jnp.float32),
    )(acc1p0, acc1p1, z, dinv16, b1.reshape(1, _F), W2.reshape(1, _F))


def _final_body(acc2t_ref, y_ref, dinv_ref, b2_ref, o_ref):
    t = (dinv_ref[:, 0:1]
         * (acc2t_ref[:, 0:1] + acc2t_ref[:, 1:2] + y_ref[...])
         + b2_ref[0, 0])
    m = jnp.max(t, axis=1, keepdims=True)
    e = jnp.exp(t - m)
    o_ref[...] = e / jnp.sum(e, axis=1, keepdims=True)


def _final(acc2pT, yprime, dinv16, b2, N):
    return pl.pallas_call(
        _final_body,
        grid=(N // _NB,),
        in_specs=[
            pl.BlockSpec((_NB, 2), lambda i: (i, 0)),
            pl.BlockSpec((_NB, 1), lambda i: (i, 0)),
            pl.BlockSpec((_NB, _F), lambda i: (i, 0)),
            pl.BlockSpec((1, 1), lambda i: (0, 0)),
        ],
        out_specs=pl.BlockSpec((_NB, 1), lambda i: (i, 0)),
        out_shape=jax.ShapeDtypeStruct((N, 1), jnp.float32),
    )(acc2pT, yprime, dinv16, b2.reshape(1, 1))

def _t2_body(i_ref, o_ref):
    o_ref[...] = jnp.transpose(i_ref[...], (1, 0))


def _transpose2(v):
    nb = 12544
    return pl.pallas_call(
        _t2_body,
        grid=(_NPAD // nb,),
        in_specs=[pl.BlockSpec((_NC, nb), lambda i: (0, i))],
        out_specs=pl.BlockSpec((nb, _NC), lambda i: (i, 0)),
        out_shape=jax.ShapeDtypeStruct((_NPAD, _NC), jnp.float32),
    )(v.reshape(_NC, _NPAD))


def kernel(data, edge_index, Wih, Whh, bih, bhh, W1, b1, W2, b2):
    N = data.shape[1]
    E = edge_index.shape[1]
    pad = _EPAD - E
    src2d = jnp.concatenate(
        [edge_index[0], jnp.zeros((pad,), jnp.int32)]).reshape(-1, _CH)
    dst2d = jnp.concatenate(
        [edge_index[1], jnp.full((pad,), N, jnp.int32)]).reshape(-1, _CH)

    deg = _sc_degree(dst2d)
    xw = _lstm_proj(data, Wih, Whh, bih, bhh, W1)

    degpT = _transpose2(deg)
    z, dinv16 = _scale(degpT, xw, N)
    a1p = _sc_edge_pass_rows(src2d, dst2d, z)
    yprime = _conv1_head(a1p[:_NPAD], a1p[_NPAD:], z, dinv16, b1, W2, N)
    a2p = _sc_edge_pass_scalar(src2d, dst2d, yprime.reshape(N))
    acc2pT = _transpose2(a2p)
    return _final(acc2pT, yprime, dinv16, b2, N)
